# baseline (speedup 1.0000x reference)
import numpy as np
import jax
import jax.numpy as jnp
from jax.experimental import pallas as pl
from jax.experimental.pallas import tpu as pltpu

_G = 64
_SUB = 512


def _sel_np(width):
    out = []
    for c in range(width):
        m = np.zeros((width * _G, _G), np.float32)
        m[width * np.arange(_G) + c, np.arange(_G)] = 1.0
        out.append(m)
    return out


_S3 = np.stack(_sel_np(3))
_S4 = np.stack(_sel_np(4))
_E3 = np.stack([m.T for m in _sel_np(3)])
_s4 = _sel_np(4)
_E4 = np.stack([_s4[0].T, (_s4[1] + _s4[2]).T, _s4[3].T])


def _body(pos_ref, quat_ref, scale_ref, rgb_ref, opa_ref, rot_ref, tran_ref,
          s3_ref, s4_ref, e3_ref, e4_ref,
          pimg_ref, rgb_o_ref, opa_o_ref, cov_ref):
    f32 = jnp.float32

    def mm(a, b):
        return jax.lax.dot_general(a, b, (((1,), (0,)), ((), ())),
                                   preferred_element_type=f32)

    m_pos = pos_ref[...]
    m_quat = quat_ref[...]
    m_scale = scale_ref[...]

    px = mm(m_pos, s3_ref[0])
    py = mm(m_pos, s3_ref[1])
    pz = mm(m_pos, s3_ref[2])
    qw = mm(m_quat, s4_ref[0])
    qx = mm(m_quat, s4_ref[1])
    qy = mm(m_quat, s4_ref[2])
    qz = mm(m_quat, s4_ref[3])
    sx = mm(m_scale, s3_ref[0])
    sy = mm(m_scale, s3_ref[1])
    sz = mm(m_scale, s3_ref[2])

    r = [[rot_ref[i, j] for j in range(3)] for i in range(3)]
    t0 = tran_ref[0]
    t1 = tran_ref[1]
    t2 = tran_ref[2]

    xc = px * r[0][0] + py * r[0][1] + pz * r[0][2] + t0
    yc = px * r[1][0] + py * r[1][1] + pz * r[1][2] + t1
    zc = px * r[2][0] + py * r[2][1] + pz * r[2][2] + t2

    zi = 1.0 / zc
    u = xc * zi
    v = yc * zi
    depth = jnp.sqrt(xc * xc + yc * yc + zc * zc)

    qn = 1.0 / (jnp.sqrt(qw * qw + qx * qx + qy * qy + qz * qz) + 1e-8)
    w = qw * qn
    x = qx * qn
    y = qy * qn
    z = qz * qn
    xx = x * x
    yy = y * y
    zz = z * z
    xy = x * y
    xz = x * z
    yz = y * z
    wx = w * x
    wy = w * y
    wz = w * z
    R00 = 1.0 - 2.0 * (yy + zz)
    R01 = 2.0 * (xy - wz)
    R02 = 2.0 * (xz + wy)
    R10 = 2.0 * (xy + wz)
    R11 = 1.0 - 2.0 * (xx + zz)
    R12 = 2.0 * (yz - wx)
    R20 = 2.0 * (xz - wy)
    R21 = 2.0 * (yz + wx)
    R22 = 1.0 - 2.0 * (xx + yy)

    ax = jnp.abs(sx) + 0.0001
    ay = jnp.abs(sy) + 0.0001
    az = jnp.abs(sz) + 0.0001

    a00 = R00 * ax
    a01 = R01 * ay
    a02 = R02 * az
    a10 = R10 * ax
    a11 = R11 * ay
    a12 = R12 * az
    a20 = R20 * ax
    a21 = R21 * ay
    a22 = R22 * az
    S00 = a00 * a00 + a01 * a01 + a02 * a02
    S01 = a00 * a10 + a01 * a11 + a02 * a12
    S02 = a00 * a20 + a01 * a21 + a02 * a22
    S11 = a10 * a10 + a11 * a11 + a12 * a12
    S12 = a10 * a20 + a11 * a21 + a12 * a22
    S22 = a20 * a20 + a21 * a21 + a22 * a22

    jw00 = zi * (r[0][0] - u * r[2][0])
    jw01 = zi * (r[0][1] - u * r[2][1])
    jw02 = zi * (r[0][2] - u * r[2][2])
    jw10 = zi * (r[1][0] - v * r[2][0])
    jw11 = zi * (r[1][1] - v * r[2][1])
    jw12 = zi * (r[1][2] - v * r[2][2])

    T00 = jw00 * S00 + jw01 * S01 + jw02 * S02
    T01 = jw00 * S01 + jw01 * S11 + jw02 * S12
    T02 = jw00 * S02 + jw01 * S12 + jw02 * S22
    T10 = jw10 * S00 + jw11 * S01 + jw12 * S02
    T11 = jw10 * S01 + jw11 * S11 + jw12 * S12
    T12 = jw10 * S02 + jw11 * S12 + jw12 * S22
    c00 = T00 * jw00 + T01 * jw01 + T02 * jw02
    c01 = T00 * jw10 + T01 * jw11 + T02 * jw12
    c11 = T10 * jw10 + T11 * jw11 + T12 * jw12

    pimg_ref[...] = mm(u, e3_ref[0]) + mm(v, e3_ref[1]) + mm(depth, e3_ref[2])
    cov_ref[...] = mm(c00, e4_ref[0]) + mm(c01, e4_ref[1]) + mm(c11, e4_ref[2])
    rgb_o_ref[...] = jax.nn.sigmoid(rgb_ref[...])
    opa_o_ref[...] = jax.nn.sigmoid(opa_ref[...])


def kernel(position, rgb_color, opacity, quaternion_rotation, scale, rot, tran):
    n = position.shape[0]
    rows = n // _G
    g = -(-rows // _SUB)

    pos_v = position.reshape(rows, 3 * _G)
    quat_v = quaternion_rotation.reshape(rows, 4 * _G)
    scale_v = scale.reshape(rows, 3 * _G)
    rgb_v = rgb_color.reshape(rows, 3 * _G)
    opa_v = opacity.reshape(rows, _G)

    out_shapes = (
        jax.ShapeDtypeStruct((rows, 3 * _G), jnp.float32),
        jax.ShapeDtypeStruct((rows, 3 * _G), jnp.float32),
        jax.ShapeDtypeStruct((rows, _G), jnp.float32),
        jax.ShapeDtypeStruct((rows, 4 * _G), jnp.float32),
    )
    grid_spec = pl.GridSpec(
        grid=(g,),
        in_specs=[
            pl.BlockSpec((_SUB, 3 * _G), lambda i: (i, 0)),
            pl.BlockSpec((_SUB, 4 * _G), lambda i: (i, 0)),
            pl.BlockSpec((_SUB, 3 * _G), lambda i: (i, 0)),
            pl.BlockSpec((_SUB, 3 * _G), lambda i: (i, 0)),
            pl.BlockSpec((_SUB, _G), lambda i: (i, 0)),
            pl.BlockSpec(memory_space=pltpu.SMEM),
            pl.BlockSpec(memory_space=pltpu.SMEM),
            pl.BlockSpec((3, 3 * _G, _G), lambda i: (0, 0, 0)),
            pl.BlockSpec((4, 4 * _G, _G), lambda i: (0, 0, 0)),
            pl.BlockSpec((3, _G, 3 * _G), lambda i: (0, 0, 0)),
            pl.BlockSpec((3, _G, 4 * _G), lambda i: (0, 0, 0)),
        ],
        out_specs=[
            pl.BlockSpec((_SUB, 3 * _G), lambda i: (i, 0)),
            pl.BlockSpec((_SUB, 3 * _G), lambda i: (i, 0)),
            pl.BlockSpec((_SUB, _G), lambda i: (i, 0)),
            pl.BlockSpec((_SUB, 4 * _G), lambda i: (i, 0)),
        ],
    )
    pimg_o, rgb_o, opa_o, cov_o = pl.pallas_call(
        _body,
        grid_spec=grid_spec,
        out_shape=out_shapes,
        compiler_params=pltpu.CompilerParams(
            dimension_semantics=("arbitrary",),
        ),
    )(pos_v, quat_v, scale_v, rgb_v, opa_v, rot, tran,
      jnp.asarray(_S3), jnp.asarray(_S4), jnp.asarray(_E3), jnp.asarray(_E4))

    pos_img = pimg_o.reshape(n, 3)
    rgb = rgb_o.reshape(n, 3)
    opa = opa_o.reshape(n, 1)
    cov_2d = cov_o.reshape(n, 2, 2)
    return pos_img, rgb, opa, cov_2d


# speedup vs baseline: 2.9046x; 2.9046x over previous
import jax
import jax.numpy as jnp
from jax.experimental import pallas as pl
from jax.experimental.pallas import tpu as pltpu

_B = 2048


def _body(pos_ref, quat_ref, scale_ref, rgb_ref, opa_ref, rot_ref, tran_ref,
          pimg_ref, rgb_o_ref, opa_o_ref, cov_ref):
    tr = jnp.transpose

    pos_t = tr(pos_ref[...])
    quat_t = tr(quat_ref[...])
    scale_t = tr(scale_ref[...])

    px = pos_t[0:1]
    py = pos_t[1:2]
    pz = pos_t[2:3]

    r = [[rot_ref[i, j] for j in range(3)] for i in range(3)]
    t0 = tran_ref[0]
    t1 = tran_ref[1]
    t2 = tran_ref[2]

    xc = px * r[0][0] + py * r[0][1] + pz * r[0][2] + t0
    yc = px * r[1][0] + py * r[1][1] + pz * r[1][2] + t1
    zc = px * r[2][0] + py * r[2][1] + pz * r[2][2] + t2

    zi = 1.0 / zc
    u = xc * zi
    v = yc * zi
    depth = jnp.sqrt(xc * xc + yc * yc + zc * zc)

    q2 = quat_t * quat_t
    qn = 1.0 / (jnp.sqrt(q2[0:1] + q2[1:2] + q2[2:3] + q2[3:4]) + 1e-8)
    w = quat_t[0:1] * qn
    x = quat_t[1:2] * qn
    y = quat_t[2:3] * qn
    z = quat_t[3:4] * qn
    xx = x * x
    yy = y * y
    zz = z * z
    xy = x * y
    xz = x * z
    yz = y * z
    wx = w * x
    wy = w * y
    wz = w * z
    R00 = 1.0 - 2.0 * (yy + zz)
    R01 = 2.0 * (xy - wz)
    R02 = 2.0 * (xz + wy)
    R10 = 2.0 * (xy + wz)
    R11 = 1.0 - 2.0 * (xx + zz)
    R12 = 2.0 * (yz - wx)
    R20 = 2.0 * (xz - wy)
    R21 = 2.0 * (yz + wx)
    R22 = 1.0 - 2.0 * (xx + yy)

    sa = jnp.abs(scale_t) + 0.0001
    ax = sa[0:1]
    ay = sa[1:2]
    az = sa[2:3]

    a00 = R00 * ax
    a01 = R01 * ay
    a02 = R02 * az
    a10 = R10 * ax
    a11 = R11 * ay
    a12 = R12 * az
    a20 = R20 * ax
    a21 = R21 * ay
    a22 = R22 * az
    S00 = a00 * a00 + a01 * a01 + a02 * a02
    S01 = a00 * a10 + a01 * a11 + a02 * a12
    S02 = a00 * a20 + a01 * a21 + a02 * a22
    S11 = a10 * a10 + a11 * a11 + a12 * a12
    S12 = a10 * a20 + a11 * a21 + a12 * a22
    S22 = a20 * a20 + a21 * a21 + a22 * a22

    jw00 = zi * (r[0][0] - u * r[2][0])
    jw01 = zi * (r[0][1] - u * r[2][1])
    jw02 = zi * (r[0][2] - u * r[2][2])
    jw10 = zi * (r[1][0] - v * r[2][0])
    jw11 = zi * (r[1][1] - v * r[2][1])
    jw12 = zi * (r[1][2] - v * r[2][2])

    T00 = jw00 * S00 + jw01 * S01 + jw02 * S02
    T01 = jw00 * S01 + jw01 * S11 + jw02 * S12
    T02 = jw00 * S02 + jw01 * S12 + jw02 * S22
    T10 = jw10 * S00 + jw11 * S01 + jw12 * S02
    T11 = jw10 * S01 + jw11 * S11 + jw12 * S12
    T12 = jw10 * S02 + jw11 * S12 + jw12 * S22
    c00 = T00 * jw00 + T01 * jw01 + T02 * jw02
    c01 = T00 * jw10 + T01 * jw11 + T02 * jw12
    c11 = T10 * jw10 + T11 * jw11 + T12 * jw12

    pimg_ref[...] = tr(jnp.concatenate([u, v, depth], axis=0))
    rgb_o_ref[...] = tr(jax.nn.sigmoid(tr(rgb_ref[...])))
    opa_o_ref[...] = tr(jax.nn.sigmoid(tr(opa_ref[...])))
    cov_ref[:, 0, :] = tr(jnp.concatenate([c00, c01], axis=0))
    cov_ref[:, 1, :] = tr(jnp.concatenate([c01, c11], axis=0))


def kernel(position, rgb_color, opacity, quaternion_rotation, scale, rot, tran):
    n = position.shape[0]
    g = -(-n // _B)

    out_shapes = (
        jax.ShapeDtypeStruct((n, 3), jnp.float32),
        jax.ShapeDtypeStruct((n, 3), jnp.float32),
        jax.ShapeDtypeStruct((n, 1), jnp.float32),
        jax.ShapeDtypeStruct((n, 2, 2), jnp.float32),
    )
    grid_spec = pl.GridSpec(
        grid=(g,),
        in_specs=[
            pl.BlockSpec((_B, 3), lambda i: (i, 0)),
            pl.BlockSpec((_B, 4), lambda i: (i, 0)),
            pl.BlockSpec((_B, 3), lambda i: (i, 0)),
            pl.BlockSpec((_B, 3), lambda i: (i, 0)),
            pl.BlockSpec((_B, 1), lambda i: (i, 0)),
            pl.BlockSpec(memory_space=pltpu.SMEM),
            pl.BlockSpec(memory_space=pltpu.SMEM),
        ],
        out_specs=[
            pl.BlockSpec((_B, 3), lambda i: (i, 0)),
            pl.BlockSpec((_B, 3), lambda i: (i, 0)),
            pl.BlockSpec((_B, 1), lambda i: (i, 0)),
            pl.BlockSpec((_B, 2, 2), lambda i: (i, 0, 0)),
        ],
    )
    return pl.pallas_call(
        _body,
        grid_spec=grid_spec,
        out_shape=out_shapes,
        compiler_params=pltpu.CompilerParams(
            dimension_semantics=("arbitrary",),
        ),
    )(position, quaternion_rotation, scale, rgb_color, opacity, rot, tran)


# speedup vs baseline: 2.9999x; 1.0328x over previous
import jax
import jax.numpy as jnp
from jax.experimental import pallas as pl
from jax.experimental.pallas import tpu as pltpu

_B = 4096


def _body(pos_ref, quat_ref, scale_ref, rgb_ref, opa_ref, rot_ref, tran_ref,
          pimg_ref, rgb_o_ref, opa_o_ref, cov_ref):
    tr = jnp.transpose

    pos_t = tr(pos_ref[...])
    quat_t = tr(quat_ref[...])
    scale_t = tr(scale_ref[...])

    px = pos_t[0:1]
    py = pos_t[1:2]
    pz = pos_t[2:3]

    r = [[rot_ref[i, j] for j in range(3)] for i in range(3)]
    t0 = tran_ref[0]
    t1 = tran_ref[1]
    t2 = tran_ref[2]

    xc = px * r[0][0] + py * r[0][1] + pz * r[0][2] + t0
    yc = px * r[1][0] + py * r[1][1] + pz * r[1][2] + t1
    zc = px * r[2][0] + py * r[2][1] + pz * r[2][2] + t2

    zi = 1.0 / zc
    u = xc * zi
    v = yc * zi
    depth = jnp.sqrt(xc * xc + yc * yc + zc * zc)

    q2 = quat_t * quat_t
    qn = 1.0 / (jnp.sqrt(q2[0:1] + q2[1:2] + q2[2:3] + q2[3:4]) + 1e-8)
    w = quat_t[0:1] * qn
    x = quat_t[1:2] * qn
    y = quat_t[2:3] * qn
    z = quat_t[3:4] * qn
    xx = x * x
    yy = y * y
    zz = z * z
    xy = x * y
    xz = x * z
    yz = y * z
    wx = w * x
    wy = w * y
    wz = w * z
    R00 = 1.0 - 2.0 * (yy + zz)
    R01 = 2.0 * (xy - wz)
    R02 = 2.0 * (xz + wy)
    R10 = 2.0 * (xy + wz)
    R11 = 1.0 - 2.0 * (xx + zz)
    R12 = 2.0 * (yz - wx)
    R20 = 2.0 * (xz - wy)
    R21 = 2.0 * (yz + wx)
    R22 = 1.0 - 2.0 * (xx + yy)

    sa = jnp.abs(scale_t) + 0.0001
    ax = sa[0:1]
    ay = sa[1:2]
    az = sa[2:3]

    a00 = R00 * ax
    a01 = R01 * ay
    a02 = R02 * az
    a10 = R10 * ax
    a11 = R11 * ay
    a12 = R12 * az
    a20 = R20 * ax
    a21 = R21 * ay
    a22 = R22 * az
    S00 = a00 * a00 + a01 * a01 + a02 * a02
    S01 = a00 * a10 + a01 * a11 + a02 * a12
    S02 = a00 * a20 + a01 * a21 + a02 * a22
    S11 = a10 * a10 + a11 * a11 + a12 * a12
    S12 = a10 * a20 + a11 * a21 + a12 * a22
    S22 = a20 * a20 + a21 * a21 + a22 * a22

    jw00 = zi * (r[0][0] - u * r[2][0])
    jw01 = zi * (r[0][1] - u * r[2][1])
    jw02 = zi * (r[0][2] - u * r[2][2])
    jw10 = zi * (r[1][0] - v * r[2][0])
    jw11 = zi * (r[1][1] - v * r[2][1])
    jw12 = zi * (r[1][2] - v * r[2][2])

    T00 = jw00 * S00 + jw01 * S01 + jw02 * S02
    T01 = jw00 * S01 + jw01 * S11 + jw02 * S12
    T02 = jw00 * S02 + jw01 * S12 + jw02 * S22
    T10 = jw10 * S00 + jw11 * S01 + jw12 * S02
    T11 = jw10 * S01 + jw11 * S11 + jw12 * S12
    T12 = jw10 * S02 + jw11 * S12 + jw12 * S22
    c00 = T00 * jw00 + T01 * jw01 + T02 * jw02
    c01 = T00 * jw10 + T01 * jw11 + T02 * jw12
    c11 = T10 * jw10 + T11 * jw11 + T12 * jw12

    pimg_ref[...] = tr(jnp.concatenate([u, v, depth], axis=0))
    rgb_o_ref[...] = tr(jax.nn.sigmoid(tr(rgb_ref[...])))
    opa_o_ref[...] = tr(jax.nn.sigmoid(tr(opa_ref[...])))
    cov_ref[:, 0, :] = tr(jnp.concatenate([c00, c01], axis=0))
    cov_ref[:, 1, :] = tr(jnp.concatenate([c01, c11], axis=0))


def kernel(position, rgb_color, opacity, quaternion_rotation, scale, rot, tran):
    n = position.shape[0]
    g = -(-n // _B)

    out_shapes = (
        jax.ShapeDtypeStruct((n, 3), jnp.float32),
        jax.ShapeDtypeStruct((n, 3), jnp.float32),
        jax.ShapeDtypeStruct((n, 1), jnp.float32),
        jax.ShapeDtypeStruct((n, 2, 2), jnp.float32),
    )
    grid_spec = pl.GridSpec(
        grid=(g,),
        in_specs=[
            pl.BlockSpec((_B, 3), lambda i: (i, 0)),
            pl.BlockSpec((_B, 4), lambda i: (i, 0)),
            pl.BlockSpec((_B, 3), lambda i: (i, 0)),
            pl.BlockSpec((_B, 3), lambda i: (i, 0)),
            pl.BlockSpec((_B, 1), lambda i: (i, 0)),
            pl.BlockSpec(memory_space=pltpu.SMEM),
            pl.BlockSpec(memory_space=pltpu.SMEM),
        ],
        out_specs=[
            pl.BlockSpec((_B, 3), lambda i: (i, 0)),
            pl.BlockSpec((_B, 3), lambda i: (i, 0)),
            pl.BlockSpec((_B, 1), lambda i: (i, 0)),
            pl.BlockSpec((_B, 2, 2), lambda i: (i, 0, 0)),
        ],
    )
    return pl.pallas_call(
        _body,
        grid_spec=grid_spec,
        out_shape=out_shapes,
        compiler_params=pltpu.CompilerParams(
            dimension_semantics=("parallel",),
        ),
    )(position, quaternion_rotation, scale, rgb_color, opacity, rot, tran)
